# baseline (speedup 1.0000x reference)
import math

import jax
import jax.numpy as jnp
from jax import lax
from jax.experimental import pallas as pl
from jax.experimental.pallas import tpu as pltpu

_INV_SQRT2 = 1.0 / math.sqrt(2.0)
_NUM_ANSWERS = 3129

_OB = 8


def _pool_kernel(feat_ref, pos_ref, lang_ref, xcat_ref, langm_ref, acc_ref):
    k = pl.program_id(1)
    nk = pl.num_programs(1)
    bb, _, f = feat_ref.shape
    o = pos_ref.shape[1]
    kp = xcat_ref.shape[1]

    valid = o - k * _OB
    mask = lax.broadcasted_iota(jnp.int32, feat_ref.shape, 1) < valid
    s = jnp.sum(jnp.where(mask, feat_ref[...], 0.0), axis=1)

    @pl.when(k == 0)
    def _init():
        acc_ref[...] = s
        langm_ref[...] = jnp.mean(lang_ref[...], axis=1).astype(jnp.bfloat16)

    @pl.when(jnp.logical_and(k > 0, k < nk - 1))
    def _accum():
        acc_ref[...] += s

    @pl.when(k == nk - 1)
    def _finish():
        mf = (acc_ref[...] + s) * (1.0 / o)
        mp = jnp.mean(pos_ref[...], axis=1)
        tail = jnp.concatenate(
            [mp, jnp.zeros((bb, kp - f - mp.shape[1]), jnp.float32)], axis=1)
        xcat_ref[:, :f] = mf.astype(jnp.bfloat16)
        xcat_ref[:, f:] = tail.astype(jnp.bfloat16)


def _head_kernel(xcat_ref, langm_ref, wvis_ref, wpool_ref, bpool_ref,
                 w1_ref, b1_ref, gamma_ref, beta_ref, w2_ref, b2_ref,
                 out_ref, hn_ref):
    k = pl.program_id(1)

    @pl.when(k == 0)
    def _prefix():
        visn = jnp.dot(xcat_ref[...], wvis_ref[...],
                       preferred_element_type=jnp.float32)
        x = visn + langm_ref[...].astype(jnp.float32)
        pooled = jnp.tanh(
            jnp.dot(x.astype(jnp.bfloat16), wpool_ref[...],
                    preferred_element_type=jnp.float32) + bpool_ref[...])
        h = jnp.dot(pooled.astype(jnp.bfloat16), w1_ref[...],
                    preferred_element_type=jnp.float32) + b1_ref[...]
        h = h * 0.5 * (1.0 + lax.erf(h * _INV_SQRT2))
        mu = jnp.mean(h, axis=-1, keepdims=True)
        var = jnp.mean((h - mu) ** 2, axis=-1, keepdims=True)
        hn = (h - mu) * lax.rsqrt(var + 1e-12) * gamma_ref[...] + beta_ref[...]
        hn_ref[...] = hn.astype(jnp.bfloat16)

    out_ref[...] = (jnp.dot(hn_ref[...], w2_ref[...],
                            preferred_element_type=jnp.float32) + b2_ref[...])


def kernel(feat, pos, lang_emb, w_vis, wpool, bpool, w1, b1, gamma, beta,
           w2, b2):
    B, O, F = feat.shape
    S = lang_emb.shape[1]
    H = wpool.shape[0]
    H2 = w1.shape[1]
    Kp = w_vis.shape[0]
    Ap = w2.shape[1]

    BB = B // 2
    nk = -(-O // _OB)
    xcat, langm = pl.pallas_call(
        _pool_kernel,
        out_shape=(jax.ShapeDtypeStruct((B, Kp), jnp.bfloat16),
                   jax.ShapeDtypeStruct((B, H), jnp.bfloat16)),
        grid=(2, nk),
        in_specs=[
            pl.BlockSpec((BB, _OB, F), lambda i, k: (i, k, 0)),
            pl.BlockSpec((BB, O, 4), lambda i, k: (i, 0, 0)),
            pl.BlockSpec((BB, S, H), lambda i, k: (i, 0, 0)),
        ],
        out_specs=(pl.BlockSpec((BB, Kp), lambda i, k: (i, 0)),
                   pl.BlockSpec((BB, H), lambda i, k: (i, 0))),
        scratch_shapes=[pltpu.VMEM((BB, F), jnp.float32)],
        compiler_params=pltpu.CompilerParams(
            dimension_semantics=("parallel", "arbitrary"),
        ),
    )(feat, pos, lang_emb)

    nt = 7
    ta = Ap // (2 * nt)
    out = pl.pallas_call(
        _head_kernel,
        out_shape=jax.ShapeDtypeStruct((B, Ap), jnp.float32),
        grid=(2, nt),
        in_specs=[
            pl.BlockSpec((B, Kp), lambda j, k: (0, 0)),
            pl.BlockSpec((B, H), lambda j, k: (0, 0)),
            pl.BlockSpec((Kp, H), lambda j, k: (0, 0)),
            pl.BlockSpec((H, H), lambda j, k: (0, 0)),
            pl.BlockSpec((1, H), lambda j, k: (0, 0)),
            pl.BlockSpec((H, H2), lambda j, k: (0, 0)),
            pl.BlockSpec((1, H2), lambda j, k: (0, 0)),
            pl.BlockSpec((1, H2), lambda j, k: (0, 0)),
            pl.BlockSpec((1, H2), lambda j, k: (0, 0)),
            pl.BlockSpec((H2, ta), lambda j, k: (0, j * nt + k)),
            pl.BlockSpec((1, ta), lambda j, k: (0, j * nt + k)),
        ],
        out_specs=pl.BlockSpec((B, ta), lambda j, k: (0, j * nt + k)),
        scratch_shapes=[pltpu.VMEM((B, H2), jnp.bfloat16)],
        compiler_params=pltpu.CompilerParams(
            dimension_semantics=("parallel", "arbitrary"),
            vmem_limit_bytes=48 * 1024 * 1024,
        ),
    )(xcat, langm, w_vis, wpool, bpool, w1, b1, gamma, beta, w2, b2)

    return out[:, :_NUM_ANSWERS]


# speedup vs baseline: 2.2595x; 2.2595x over previous
import math

import jax
import jax.numpy as jnp
from jax import lax
from jax.experimental import pallas as pl
from jax.experimental.pallas import tpu as pltpu

_INV_SQRT2 = 1.0 / math.sqrt(2.0)
_NUM_ANSWERS = 3129

_OB = 8


def _pool_kernel(feat_ref, pos_ref, lang_ref, xcat_ref, langm_ref, acc_ref):
    k = pl.program_id(1)
    nk = pl.num_programs(1)
    bb, _, f = feat_ref.shape
    o = pos_ref.shape[1]
    kp = xcat_ref.shape[1]

    valid = o - k * _OB
    mask = lax.broadcasted_iota(jnp.int32, feat_ref.shape, 1) < valid
    s = jnp.sum(jnp.where(mask, feat_ref[...], 0.0), axis=1)

    @pl.when(k == 0)
    def _init():
        acc_ref[...] = s
        langm_ref[...] = jnp.mean(lang_ref[...], axis=1).astype(jnp.bfloat16)

    @pl.when(jnp.logical_and(k > 0, k < nk - 1))
    def _accum():
        acc_ref[...] += s

    @pl.when(k == nk - 1)
    def _finish():
        mf = (acc_ref[...] + s) * (1.0 / o)
        mp = jnp.mean(pos_ref[...], axis=1)
        tail = jnp.concatenate(
            [mp, jnp.zeros((bb, kp - f - mp.shape[1]), jnp.float32)], axis=1)
        xcat_ref[:, :f] = mf.astype(jnp.bfloat16)
        xcat_ref[:, f:] = tail.astype(jnp.bfloat16)


def _head_kernel(xcat_ref, langm_ref, wvis_ref, wpool_ref, bpool_ref,
                 w1_ref, b1_ref, gamma_ref, beta_ref, w2_ref, b2_ref,
                 out_ref, hn_ref):
    k = pl.program_id(1)

    @pl.when(k == 0)
    def _prefix():
        visn = jnp.dot(xcat_ref[...], wvis_ref[...],
                       preferred_element_type=jnp.float32)
        x = visn + langm_ref[...].astype(jnp.float32)
        pooled = jnp.tanh(
            jnp.dot(x.astype(jnp.bfloat16), wpool_ref[...],
                    preferred_element_type=jnp.float32) + bpool_ref[...])
        h = jnp.dot(pooled.astype(jnp.bfloat16), w1_ref[...],
                    preferred_element_type=jnp.float32) + b1_ref[...]
        h = h * 0.5 * (1.0 + lax.erf(h * _INV_SQRT2))
        mu = jnp.mean(h, axis=-1, keepdims=True)
        var = jnp.mean((h - mu) ** 2, axis=-1, keepdims=True)
        hn = (h - mu) * lax.rsqrt(var + 1e-12) * gamma_ref[...] + beta_ref[...]
        hn_ref[...] = hn.astype(jnp.bfloat16)

    out_ref[...] = (jnp.dot(hn_ref[...], w2_ref[...],
                            preferred_element_type=jnp.float32) + b2_ref[...])


def kernel(feat, pos, lang_emb, w_vis, wpool, bpool, w1, b1, gamma, beta,
           w2, b2):
    B, O, F = feat.shape
    S = lang_emb.shape[1]
    H = wpool.shape[0]
    H2 = w1.shape[1]
    Kp = w_vis.shape[0]
    Ap = w2.shape[1]

    mean_feat = jnp.mean(feat, axis=1)
    mean_pos = jnp.mean(pos, axis=1)
    xcat = jnp.concatenate(
        [mean_feat, mean_pos,
         jnp.zeros((B, Kp - F - 4), jnp.float32)], axis=-1).astype(jnp.bfloat16)
    langm = jnp.mean(lang_emb, axis=1).astype(jnp.bfloat16)

    nt = 7
    ta = Ap // (2 * nt)
    out = pl.pallas_call(
        _head_kernel,
        out_shape=jax.ShapeDtypeStruct((B, Ap), jnp.float32),
        grid=(2, nt),
        in_specs=[
            pl.BlockSpec((B, Kp), lambda j, k: (0, 0)),
            pl.BlockSpec((B, H), lambda j, k: (0, 0)),
            pl.BlockSpec((Kp, H), lambda j, k: (0, 0)),
            pl.BlockSpec((H, H), lambda j, k: (0, 0)),
            pl.BlockSpec((1, H), lambda j, k: (0, 0)),
            pl.BlockSpec((H, H2), lambda j, k: (0, 0)),
            pl.BlockSpec((1, H2), lambda j, k: (0, 0)),
            pl.BlockSpec((1, H2), lambda j, k: (0, 0)),
            pl.BlockSpec((1, H2), lambda j, k: (0, 0)),
            pl.BlockSpec((H2, ta), lambda j, k: (0, j * nt + k)),
            pl.BlockSpec((1, ta), lambda j, k: (0, j * nt + k)),
        ],
        out_specs=pl.BlockSpec((B, ta), lambda j, k: (0, j * nt + k)),
        scratch_shapes=[pltpu.VMEM((B, H2), jnp.bfloat16)],
        compiler_params=pltpu.CompilerParams(
            dimension_semantics=("parallel", "arbitrary"),
            vmem_limit_bytes=48 * 1024 * 1024,
        ),
    )(xcat, langm, w_vis, wpool, bpool, w1, b1, gamma, beta, w2, b2)

    return out[:, :_NUM_ANSWERS]
